# baseline (speedup 1.0000x reference)
import functools

import jax
import jax.numpy as jnp
from jax import lax
from jax.experimental import pallas as pl
from jax.experimental.pallas import tpu as pltpu
from jax.experimental.pallas import tpu_sc as plsc

N = 10000
NP = 10240
E = 320000
D = 128
NC = 2
NS = 16
TILES = NC * NS
CH = 128
NCH = 80
GCH = 40
EP = TILES * NCH * CH
RPT = NP // NS
DH = D // 2
BR = 1024

_f32 = jnp.float32


def _mesh():
    return plsc.VectorSubcoreMesh(core_axis_name="c", subcore_axis_name="s",
                                  num_cores=NC, num_subcores=NS)


def _sc_deg_body(dst_hbm, out_hbm, hist, didx):
    c = lax.axis_index("c")
    s = lax.axis_index("s")
    t = c * NS + s
    ept = NCH * CH
    pltpu.sync_copy(dst_hbm.at[pl.ds(t * ept, ept)], didx)

    def zero(i, carry):
        hist[pl.ds(i * 16, 16)] = jnp.zeros((16,), _f32)
        return carry

    lax.fori_loop(0, NP // 16, zero, 0)

    def body(v, carry):
        idx = didx[pl.ds(v * 16, 16)]
        cnt, last = plsc.scan_count(idx)
        plsc.addupdate_scatter(hist, [idx], cnt.astype(_f32), mask=last)
        return carry

    lax.fori_loop(0, ept // 16, body, 0)
    pltpu.sync_copy(hist, out_hbm.at[pl.ds(t * NP, NP)])


def _sc_deg(dst_flat):
    return pl.kernel(
        _sc_deg_body,
        out_type=jax.ShapeDtypeStruct((TILES * NP,), _f32),
        mesh=_mesh(),
        compiler_params=pltpu.CompilerParams(needs_layout_passes=False),
        scratch_types=[
            pltpu.VMEM((NP,), _f32),
            pltpu.VMEM((NCH * CH,), jnp.int32),
        ],
    )(dst_flat)


def _sc_agg_body(hpa_hbm, hpb_hbm, src_hbm, dst_hbm, zeros_hbm, out_hbm,
                 htab, acc, sidx, didx, rows, gsem, ssem):
    c = lax.axis_index("c")
    s = lax.axis_index("s")
    t = c * NS + s
    r0 = s * RPT

    for p, hp_hbm in enumerate((hpa_hbm, hpb_hbm)):
        pltpu.sync_copy(hp_hbm.at[pl.ds(r0, RPT)], htab.at[pl.ds(r0, RPT)])

        @pl.when(c == 0)
        def _():
            pltpu.sync_copy(hp_hbm.at[pl.ds(r0, RPT)], acc.at[pl.ds(r0, RPT)])

        @pl.when(c != 0)
        def _():
            pltpu.sync_copy(zeros_hbm.at[pl.ds(r0, RPT)],
                            acc.at[pl.ds(r0, RPT)])

        plsc.subcore_barrier()

        for g in range(NCH // GCH):
            pltpu.sync_copy(src_hbm.at[pl.ds(t * NCH + g * GCH, GCH)], sidx)
            pltpu.sync_copy(dst_hbm.at[pl.ds(t * NCH + g * GCH, GCH)], didx)
            pltpu.async_copy(htab.at[sidx.at[0]], rows.at[0], gsem.at[0])

            def body(j, carry):
                b = lax.rem(j, 2)
                pltpu.make_async_copy(htab.at[sidx.at[j]], rows.at[b],
                                      gsem.at[b]).wait()
                pltpu.async_copy(rows.at[b], acc.at[didx.at[j]], ssem.at[b],
                                 add=True)

                @pl.when(jnp.logical_and(j >= 1, j + 1 < GCH))
                def _():
                    pltpu.make_async_copy(rows.at[1 - b],
                                          acc.at[didx.at[j - 1]],
                                          ssem.at[1 - b]).wait()

                @pl.when(j + 1 < GCH)
                def _():
                    pltpu.async_copy(htab.at[sidx.at[j + 1]], rows.at[1 - b],
                                     gsem.at[1 - b])

                return carry

            lax.fori_loop(0, GCH, body, 0)
            pltpu.make_async_copy(rows.at[GCH % 2], acc.at[didx.at[GCH - 2]],
                                  ssem.at[GCH % 2]).wait()
            pltpu.make_async_copy(rows.at[1 - GCH % 2],
                                  acc.at[didx.at[GCH - 1]],
                                  ssem.at[1 - GCH % 2]).wait()
        plsc.subcore_barrier()
        pltpu.sync_copy(acc.at[pl.ds(r0, RPT)],
                        out_hbm.at[c, p, pl.ds(r0, RPT)])


def _sc_agg(hpa, hpb, src2, dst2, zeros64):
    return pl.kernel(
        _sc_agg_body,
        out_type=jax.ShapeDtypeStruct((NC, 2, NP, DH), _f32),
        mesh=_mesh(),
        compiler_params=pltpu.CompilerParams(use_tc_tiling_on_sc=False),
        scratch_types=[
            pltpu.VMEM_SHARED((NP, DH), _f32),
            pltpu.VMEM_SHARED((NP, DH), _f32),
            pltpu.VMEM((GCH, CH), jnp.int32),
            pltpu.VMEM((GCH, CH), jnp.int32),
            pltpu.VMEM((2, CH, DH), _f32),
            pltpu.SemaphoreType.DMA((2,)),
            pltpu.SemaphoreType.DMA((2,)),
        ],
    )(hpa, hpb, src2, dst2, zeros64)


def _tc_degsum_body(h_ref, out_ref):
    out_ref[...] = jnp.sum(h_ref[...], axis=0, keepdims=True)


def _tc_degsum(hists):
    bc = 2048
    return pl.pallas_call(
        _tc_degsum_body,
        grid=(NP // bc,),
        in_specs=[pl.BlockSpec((TILES, bc), lambda i: (0, i))],
        out_specs=pl.BlockSpec((1, bc), lambda i: (0, i)),
        out_shape=jax.ShapeDtypeStruct((1, NP), _f32),
    )(hists)


def _tc_prep_body(x_ref, w_ref, deg_ref, hpa_ref, hpb_ref, dis_ref):
    dis = lax.rsqrt(deg_ref[...] + 1.0)
    h = jnp.dot(x_ref[...], w_ref[...], preferred_element_type=_f32) * dis
    hpa_ref[...] = h[:, :DH]
    hpb_ref[...] = h[:, DH:]
    dis_ref[...] = dis


def _tc_prep(x_p, w1, deg2):
    return pl.pallas_call(
        _tc_prep_body,
        grid=(NP // BR,),
        in_specs=[
            pl.BlockSpec((BR, D), lambda i: (i, 0)),
            pl.BlockSpec((D, D), lambda i: (0, 0)),
            pl.BlockSpec((BR, 1), lambda i: (i, 0)),
        ],
        out_specs=[
            pl.BlockSpec((BR, DH), lambda i: (i, 0)),
            pl.BlockSpec((BR, DH), lambda i: (i, 0)),
            pl.BlockSpec((BR, 1), lambda i: (i, 0)),
        ],
        out_shape=[
            jax.ShapeDtypeStruct((NP, DH), _f32),
            jax.ShapeDtypeStruct((NP, DH), _f32),
            jax.ShapeDtypeStruct((NP, 1), _f32),
        ],
    )(x_p, w1, deg2)


def _p_specs():
    return [
        pl.BlockSpec((1, 1, BR, DH), lambda i, c=c, p=p: (c, p, i, 0))
        for c in range(NC) for p in range(2)
    ]


def _combine(p0a, p0b, p1a, p1b, dis, b):
    lo = p0a[0, 0] + p1a[0, 0]
    hi = p0b[0, 0] + p1b[0, 0]
    return jnp.concatenate([lo, hi], axis=1) * dis + b


def _tc_mid_body(p0a, p0b, p1a, p1b, dis_ref, b_ref, w_ref,
                 hpa_ref, hpb_ref):
    dis = dis_ref[...]
    xn = jnp.maximum(_combine(p0a, p0b, p1a, p1b, dis, b_ref[...]), 0.0)
    h = jnp.dot(xn, w_ref[...], preferred_element_type=_f32) * dis
    hpa_ref[...] = h[:, :DH]
    hpb_ref[...] = h[:, DH:]


def _tc_mid(p, dis, b, w):
    return pl.pallas_call(
        _tc_mid_body,
        grid=(NP // BR,),
        in_specs=_p_specs() + [
            pl.BlockSpec((BR, 1), lambda i: (i, 0)),
            pl.BlockSpec((1, D), lambda i: (0, 0)),
            pl.BlockSpec((D, D), lambda i: (0, 0)),
        ],
        out_specs=[
            pl.BlockSpec((BR, DH), lambda i: (i, 0)),
            pl.BlockSpec((BR, DH), lambda i: (i, 0)),
        ],
        out_shape=[
            jax.ShapeDtypeStruct((NP, DH), _f32),
            jax.ShapeDtypeStruct((NP, DH), _f32),
        ],
    )(p, p, p, p, dis, b, w)


def _tc_fin_body(p0a, p0b, p1a, p1b, dis_ref, b_ref, out_ref):
    out_ref[...] = _combine(p0a, p0b, p1a, p1b, dis_ref[...], b_ref[...])


def _tc_fin(p, dis, b):
    return pl.pallas_call(
        _tc_fin_body,
        grid=(NP // BR,),
        in_specs=_p_specs() + [
            pl.BlockSpec((BR, 1), lambda i: (i, 0)),
            pl.BlockSpec((1, D), lambda i: (0, 0)),
        ],
        out_specs=pl.BlockSpec((BR, D), lambda i: (i, 0)),
        out_shape=jax.ShapeDtypeStruct((NP, D), _f32),
    )(p, p, p, p, dis, b)


def kernel(x, edge_index, W1, b1, W2, b2, W3, b3):
    src = edge_index[0].astype(jnp.int32)
    dst = edge_index[1].astype(jnp.int32)
    pad = N + (jnp.arange(EP - E, dtype=jnp.int32) % (NP - N))
    src_flat = jnp.concatenate([src, pad])
    dst_flat = jnp.concatenate([dst, pad])
    src2 = src_flat.reshape(TILES * NCH, CH)
    dst2 = dst_flat.reshape(TILES * NCH, CH)

    x_p = jnp.pad(x, ((0, NP - N), (0, 0)))
    zeros64 = jnp.zeros((NP, DH), _f32)
    b1r = b1.reshape(1, D)
    b2r = b2.reshape(1, D)
    b3r = b3.reshape(1, D)

    hists = _sc_deg(dst_flat)
    deg2 = _tc_degsum(hists.reshape(TILES, NP)).reshape(NP, 1)
    hp1a, hp1b, dis = _tc_prep(x_p, W1, deg2)
    p1 = _sc_agg(hp1a, hp1b, src2, dst2, zeros64)
    hp2a, hp2b = _tc_mid(p1, dis, b1r, W2)
    p2 = _sc_agg(hp2a, hp2b, src2, dst2, zeros64)
    hp3a, hp3b = _tc_mid(p2, dis, b2r, W3)
    p3 = _sc_agg(hp3a, hp3b, src2, dst2, zeros64)
    out = _tc_fin(p3, dis, b3r)
    return out[:N]


# speedup vs baseline: 1.2872x; 1.2872x over previous
import functools

import jax
import jax.numpy as jnp
from jax import lax
from jax.experimental import pallas as pl
from jax.experimental.pallas import tpu as pltpu
from jax.experimental.pallas import tpu_sc as plsc

N = 10000
NP = 10240
E = 320000
D = 128
NC = 2
NS = 16
TILES = NC * NS
CH = 128
NCH = 80
GCH = 40
EP = TILES * NCH * CH
RPT = NP // NS
BR = 1000

_f32 = jnp.float32


def _mesh():
    return plsc.VectorSubcoreMesh(core_axis_name="c", subcore_axis_name="s",
                                  num_cores=NC, num_subcores=NS)


def _sc_deg_body(dst_hbm, out_hbm, hist, didx):
    c = lax.axis_index("c")
    s = lax.axis_index("s")
    t = c * NS + s
    ept = NCH * CH
    pltpu.sync_copy(dst_hbm.at[pl.ds(t * ept, ept)], didx)

    def zero(i, carry):
        hist[pl.ds(i * 16, 16)] = jnp.zeros((16,), _f32)
        return carry

    lax.fori_loop(0, NP // 16, zero, 0)

    def body(v, carry):
        idx = didx[pl.ds(v * 16, 16)]
        cnt, last = plsc.scan_count(idx)
        plsc.addupdate_scatter(hist, [idx], cnt.astype(_f32), mask=last)
        return carry

    lax.fori_loop(0, ept // 16, body, 0)
    pltpu.sync_copy(hist, out_hbm.at[pl.ds(t * NP, NP)])


def _sc_deg(dst_flat):
    return pl.kernel(
        _sc_deg_body,
        out_type=jax.ShapeDtypeStruct((TILES * NP,), _f32),
        mesh=_mesh(),
        compiler_params=pltpu.CompilerParams(needs_layout_passes=False),
        scratch_types=[
            pltpu.VMEM((NP,), _f32),
            pltpu.VMEM((NCH * CH,), jnp.int32),
        ],
    )(dst_flat)


def _sc_agg_body(hp_hbm, src_hbm, dst_hbm, zeros_hbm, out_hbm,
                 acc, sidx, didx, rows, gsem, ssem):
    c = lax.axis_index("c")
    s = lax.axis_index("s")
    t = c * NS + s
    r0 = s * RPT

    @pl.when(c == 0)
    def _():
        pltpu.sync_copy(hp_hbm.at[pl.ds(r0, RPT)], acc.at[pl.ds(r0, RPT)])

    @pl.when(c != 0)
    def _():
        pltpu.sync_copy(zeros_hbm.at[pl.ds(r0, RPT)], acc.at[pl.ds(r0, RPT)])

    plsc.subcore_barrier()

    for g in range(NCH // GCH):
        pltpu.sync_copy(src_hbm.at[pl.ds(t * NCH + g * GCH, GCH)], sidx)
        pltpu.sync_copy(dst_hbm.at[pl.ds(t * NCH + g * GCH, GCH)], didx)
        pltpu.async_copy(hp_hbm.at[sidx.at[0]], rows.at[0], gsem.at[0])

        def body(j, carry):
            b = lax.rem(j, 2)
            pltpu.make_async_copy(hp_hbm.at[sidx.at[j]], rows.at[b],
                                  gsem.at[b]).wait()
            pltpu.async_copy(rows.at[b], acc.at[didx.at[j]], ssem.at[b],
                             add=True)

            @pl.when(jnp.logical_and(j >= 1, j + 1 < GCH))
            def _():
                pltpu.make_async_copy(rows.at[1 - b],
                                      acc.at[didx.at[j - 1]],
                                      ssem.at[1 - b]).wait()

            @pl.when(j + 1 < GCH)
            def _():
                pltpu.async_copy(hp_hbm.at[sidx.at[j + 1]], rows.at[1 - b],
                                 gsem.at[1 - b])

            return carry

        lax.fori_loop(0, GCH, body, 0)
        pltpu.make_async_copy(rows.at[GCH % 2], acc.at[didx.at[GCH - 2]],
                              ssem.at[GCH % 2]).wait()
        pltpu.make_async_copy(rows.at[1 - GCH % 2], acc.at[didx.at[GCH - 1]],
                              ssem.at[1 - GCH % 2]).wait()
    plsc.subcore_barrier()
    pltpu.sync_copy(acc.at[pl.ds(r0, RPT)], out_hbm.at[c, pl.ds(r0, RPT)])


def _sc_agg(hp, src2, dst2, zeros128):
    return pl.kernel(
        _sc_agg_body,
        out_type=jax.ShapeDtypeStruct((NC, NP, D), _f32),
        mesh=_mesh(),
        scratch_types=[
            pltpu.VMEM_SHARED((NP, D), _f32),
            pltpu.VMEM((GCH, CH), jnp.int32),
            pltpu.VMEM((GCH, CH), jnp.int32),
            pltpu.VMEM((2, CH, D), _f32),
            pltpu.SemaphoreType.DMA((2,)),
            pltpu.SemaphoreType.DMA((2,)),
        ],
    )(hp, src2, dst2, zeros128)


def _tc_degsum_body(h_ref, out_ref):
    out_ref[...] = jnp.sum(h_ref[...], axis=0, keepdims=True)


def _tc_degsum(hists):
    bc = 2048
    return pl.pallas_call(
        _tc_degsum_body,
        grid=(NP // bc,),
        in_specs=[pl.BlockSpec((TILES, bc), lambda i: (0, i))],
        out_specs=pl.BlockSpec((1, bc), lambda i: (0, i)),
        out_shape=jax.ShapeDtypeStruct((1, NP), _f32),
    )(hists)


def _tc_prep_body(x_ref, w_ref, deg_ref, hp_ref, dis_ref):
    dis = lax.rsqrt(deg_ref[...] + 1.0)
    h = jnp.dot(x_ref[...], w_ref[...], preferred_element_type=_f32)
    hp_ref[...] = h * dis
    dis_ref[...] = dis


def _tc_prep(x, w1, deg2):
    return pl.pallas_call(
        _tc_prep_body,
        grid=(N // BR,),
        in_specs=[
            pl.BlockSpec((BR, D), lambda i: (i, 0)),
            pl.BlockSpec((D, D), lambda i: (0, 0)),
            pl.BlockSpec((BR, 1), lambda i: (i, 0)),
        ],
        out_specs=[
            pl.BlockSpec((BR, D), lambda i: (i, 0)),
            pl.BlockSpec((BR, 1), lambda i: (i, 0)),
        ],
        out_shape=[
            jax.ShapeDtypeStruct((NP, D), _f32),
            jax.ShapeDtypeStruct((NP, 1), _f32),
        ],
    )(x, w1, deg2)


def _tc_mid_body(pa_ref, pb_ref, dis_ref, b_ref, w_ref, hp_ref):
    dis = dis_ref[...]
    xn = jnp.maximum((pa_ref[0] + pb_ref[0]) * dis + b_ref[...], 0.0)
    hp_ref[...] = jnp.dot(xn, w_ref[...], preferred_element_type=_f32) * dis


def _tc_mid(p, dis, b, w):
    return pl.pallas_call(
        _tc_mid_body,
        grid=(N // BR,),
        in_specs=[
            pl.BlockSpec((1, BR, D), lambda i: (0, i, 0)),
            pl.BlockSpec((1, BR, D), lambda i: (1, i, 0)),
            pl.BlockSpec((BR, 1), lambda i: (i, 0)),
            pl.BlockSpec((1, D), lambda i: (0, 0)),
            pl.BlockSpec((D, D), lambda i: (0, 0)),
        ],
        out_specs=pl.BlockSpec((BR, D), lambda i: (i, 0)),
        out_shape=jax.ShapeDtypeStruct((NP, D), _f32),
    )(p, p, dis, b, w)


def _tc_fin_body(pa_ref, pb_ref, dis_ref, b_ref, out_ref):
    out_ref[...] = (pa_ref[0] + pb_ref[0]) * dis_ref[...] + b_ref[...]


def _tc_fin(p, dis, b):
    return pl.pallas_call(
        _tc_fin_body,
        grid=(N // BR,),
        in_specs=[
            pl.BlockSpec((1, BR, D), lambda i: (0, i, 0)),
            pl.BlockSpec((1, BR, D), lambda i: (1, i, 0)),
            pl.BlockSpec((BR, 1), lambda i: (i, 0)),
            pl.BlockSpec((1, D), lambda i: (0, 0)),
        ],
        out_specs=pl.BlockSpec((BR, D), lambda i: (i, 0)),
        out_shape=jax.ShapeDtypeStruct((N, D), _f32),
    )(p, p, dis, b)


def kernel(x, edge_index, W1, b1, W2, b2, W3, b3):
    src = edge_index[0].astype(jnp.int32)
    dst = edge_index[1].astype(jnp.int32)
    pad = N + (jnp.arange(EP - E, dtype=jnp.int32) % (NP - N))
    src_flat = jnp.concatenate([src, pad])
    dst_flat = jnp.concatenate([dst, pad])
    src2 = src_flat.reshape(TILES * NCH, CH)
    dst2 = dst_flat.reshape(TILES * NCH, CH)

    zeros128 = jnp.zeros((NP, D), _f32)
    b1r = b1.reshape(1, D)
    b2r = b2.reshape(1, D)
    b3r = b3.reshape(1, D)

    hists = _sc_deg(dst_flat)
    deg2 = _tc_degsum(hists.reshape(TILES, NP)).reshape(NP, 1)
    hp1, dis = _tc_prep(x, W1, deg2)
    p1 = _sc_agg(hp1, src2, dst2, zeros128)
    hp2 = _tc_mid(p1, dis, b1r, W2)
    p2 = _sc_agg(hp2, src2, dst2, zeros128)
    hp3 = _tc_mid(p2, dis, b2r, W3)
    p3 = _sc_agg(hp3, src2, dst2, zeros128)
    return _tc_fin(p3, dis, b3r)


# speedup vs baseline: 1.2946x; 1.0058x over previous
import functools

import jax
import jax.numpy as jnp
from jax import lax
from jax.experimental import pallas as pl
from jax.experimental.pallas import tpu as pltpu
from jax.experimental.pallas import tpu_sc as plsc

N = 10000
NP = 10240
E = 320000
D = 128
NC = 2
NS = 16
TILES = NC * NS
CH = 128
NCH = 80
GCH = 40
EP = TILES * NCH * CH
RPT = NP // NS
BR = 1000

_f32 = jnp.float32


def _mesh():
    return plsc.VectorSubcoreMesh(core_axis_name="c", subcore_axis_name="s",
                                  num_cores=NC, num_subcores=NS)


def _sc_deg_body(dst_hbm, out_hbm, hist, didx):
    c = lax.axis_index("c")
    s = lax.axis_index("s")
    t = c * NS + s
    ept = NCH * CH
    pltpu.sync_copy(dst_hbm.at[pl.ds(t * ept, ept)], didx)

    def zero(i, carry):
        hist[pl.ds(i * 16, 16)] = jnp.zeros((16,), _f32)
        return carry

    lax.fori_loop(0, NP // 16, zero, 0)

    def body(v, carry):
        idx = didx[pl.ds(v * 16, 16)]
        cnt, last = plsc.scan_count(idx)
        plsc.addupdate_scatter(hist, [idx], cnt.astype(_f32), mask=last)
        return carry

    lax.fori_loop(0, ept // 16, body, 0)
    pltpu.sync_copy(hist, out_hbm.at[pl.ds(t * NP, NP)])


def _sc_deg(dst_flat):
    return pl.kernel(
        _sc_deg_body,
        out_type=jax.ShapeDtypeStruct((TILES * NP,), _f32),
        mesh=_mesh(),
        compiler_params=pltpu.CompilerParams(needs_layout_passes=False),
        scratch_types=[
            pltpu.VMEM((NP,), _f32),
            pltpu.VMEM((NCH * CH,), jnp.int32),
        ],
    )(dst_flat)


def _sc_agg_body(hp_hbm, src_hbm, dst_hbm, zeros_hbm, out_hbm,
                 acc, sidx, didx, rows, gsem, ssem):
    c = lax.axis_index("c")
    s = lax.axis_index("s")
    t = c * NS + s
    r0 = s * RPT

    @pl.when(c == 0)
    def _():
        pltpu.sync_copy(hp_hbm.at[pl.ds(r0, RPT)], acc.at[pl.ds(r0, RPT)])

    @pl.when(c != 0)
    def _():
        pltpu.sync_copy(zeros_hbm.at[pl.ds(r0, RPT)], acc.at[pl.ds(r0, RPT)])

    plsc.subcore_barrier()

    for g in range(NCH // GCH):
        pltpu.sync_copy(src_hbm.at[pl.ds(t * NCH + g * GCH, GCH)], sidx)
        pltpu.sync_copy(dst_hbm.at[pl.ds(t * NCH + g * GCH, GCH)], didx)
        pltpu.async_copy(hp_hbm.at[sidx.at[0]], rows.at[0], gsem.at[0])

        def body(j, carry):
            b = lax.rem(j, 2)
            pltpu.make_async_copy(hp_hbm.at[sidx.at[j]], rows.at[b],
                                  gsem.at[b]).wait()

            @pl.when(j >= 1)
            def _():
                pltpu.make_async_copy(rows.at[1 - b],
                                      acc.at[didx.at[j - 1]],
                                      ssem.at[1 - b]).wait()

            @pl.when(j + 1 < GCH)
            def _():
                pltpu.async_copy(hp_hbm.at[sidx.at[j + 1]], rows.at[1 - b],
                                 gsem.at[1 - b])

            pltpu.async_copy(rows.at[b], acc.at[didx.at[j]], ssem.at[b],
                             add=True)
            return carry

        lax.fori_loop(0, GCH, body, 0)
        pltpu.make_async_copy(rows.at[1 - GCH % 2], acc.at[didx.at[GCH - 1]],
                              ssem.at[1 - GCH % 2]).wait()
    plsc.subcore_barrier()
    pltpu.sync_copy(acc.at[pl.ds(r0, RPT)], out_hbm.at[c, pl.ds(r0, RPT)])


def _sc_agg(hp, src2, dst2, zeros128):
    return pl.kernel(
        _sc_agg_body,
        out_type=jax.ShapeDtypeStruct((NC, NP, D), _f32),
        mesh=_mesh(),
        scratch_types=[
            pltpu.VMEM_SHARED((NP, D), _f32),
            pltpu.VMEM((GCH, CH), jnp.int32),
            pltpu.VMEM((GCH, CH), jnp.int32),
            pltpu.VMEM((2, CH, D), _f32),
            pltpu.SemaphoreType.DMA((2,)),
            pltpu.SemaphoreType.DMA((2,)),
        ],
    )(hp, src2, dst2, zeros128)


def _tc_degsum_body(h_ref, out_ref):
    out_ref[...] = jnp.sum(h_ref[...], axis=0, keepdims=True)


def _tc_degsum(hists):
    bc = 2048
    return pl.pallas_call(
        _tc_degsum_body,
        grid=(NP // bc,),
        in_specs=[pl.BlockSpec((TILES, bc), lambda i: (0, i))],
        out_specs=pl.BlockSpec((1, bc), lambda i: (0, i)),
        out_shape=jax.ShapeDtypeStruct((1, NP), _f32),
    )(hists)


def _tc_prep_body(x_ref, w_ref, deg_ref, hp_ref, dis_ref):
    dis = lax.rsqrt(deg_ref[...] + 1.0)
    h = jnp.dot(x_ref[...], w_ref[...], preferred_element_type=_f32)
    hp_ref[...] = h * dis
    dis_ref[...] = dis


def _tc_prep(x, w1, deg2):
    return pl.pallas_call(
        _tc_prep_body,
        grid=(N // BR,),
        in_specs=[
            pl.BlockSpec((BR, D), lambda i: (i, 0)),
            pl.BlockSpec((D, D), lambda i: (0, 0)),
            pl.BlockSpec((BR, 1), lambda i: (i, 0)),
        ],
        out_specs=[
            pl.BlockSpec((BR, D), lambda i: (i, 0)),
            pl.BlockSpec((BR, 1), lambda i: (i, 0)),
        ],
        out_shape=[
            jax.ShapeDtypeStruct((NP, D), _f32),
            jax.ShapeDtypeStruct((NP, 1), _f32),
        ],
    )(x, w1, deg2)


def _tc_mid_body(pa_ref, pb_ref, dis_ref, b_ref, w_ref, hp_ref):
    dis = dis_ref[...]
    xn = jnp.maximum((pa_ref[0] + pb_ref[0]) * dis + b_ref[...], 0.0)
    hp_ref[...] = jnp.dot(xn, w_ref[...], preferred_element_type=_f32) * dis


def _tc_mid(p, dis, b, w):
    return pl.pallas_call(
        _tc_mid_body,
        grid=(N // BR,),
        in_specs=[
            pl.BlockSpec((1, BR, D), lambda i: (0, i, 0)),
            pl.BlockSpec((1, BR, D), lambda i: (1, i, 0)),
            pl.BlockSpec((BR, 1), lambda i: (i, 0)),
            pl.BlockSpec((1, D), lambda i: (0, 0)),
            pl.BlockSpec((D, D), lambda i: (0, 0)),
        ],
        out_specs=pl.BlockSpec((BR, D), lambda i: (i, 0)),
        out_shape=jax.ShapeDtypeStruct((NP, D), _f32),
    )(p, p, dis, b, w)


def _tc_fin_body(pa_ref, pb_ref, dis_ref, b_ref, out_ref):
    out_ref[...] = (pa_ref[0] + pb_ref[0]) * dis_ref[...] + b_ref[...]


def _tc_fin(p, dis, b):
    return pl.pallas_call(
        _tc_fin_body,
        grid=(N // BR,),
        in_specs=[
            pl.BlockSpec((1, BR, D), lambda i: (0, i, 0)),
            pl.BlockSpec((1, BR, D), lambda i: (1, i, 0)),
            pl.BlockSpec((BR, 1), lambda i: (i, 0)),
            pl.BlockSpec((1, D), lambda i: (0, 0)),
        ],
        out_specs=pl.BlockSpec((BR, D), lambda i: (i, 0)),
        out_shape=jax.ShapeDtypeStruct((N, D), _f32),
    )(p, p, dis, b)


def kernel(x, edge_index, W1, b1, W2, b2, W3, b3):
    src = edge_index[0].astype(jnp.int32)
    dst = edge_index[1].astype(jnp.int32)
    pad = N + (jnp.arange(EP - E, dtype=jnp.int32) % (NP - N))
    src_flat = jnp.concatenate([src, pad])
    dst_flat = jnp.concatenate([dst, pad])
    src2 = src_flat.reshape(TILES * NCH, CH)
    dst2 = dst_flat.reshape(TILES * NCH, CH)

    zeros128 = jnp.zeros((NP, D), _f32)
    b1r = b1.reshape(1, D)
    b2r = b2.reshape(1, D)
    b3r = b3.reshape(1, D)

    hists = _sc_deg(dst_flat)
    deg2 = _tc_degsum(hists.reshape(TILES, NP)).reshape(NP, 1)
    hp1, dis = _tc_prep(x, W1, deg2)
    p1 = _sc_agg(hp1, src2, dst2, zeros128)
    hp2 = _tc_mid(p1, dis, b1r, W2)
    p2 = _sc_agg(hp2, src2, dst2, zeros128)
    hp3 = _tc_mid(p2, dis, b2r, W3)
    p3 = _sc_agg(hp3, src2, dst2, zeros128)
    return _tc_fin(p3, dis, b3r)


# speedup vs baseline: 1.2947x; 1.0001x over previous
import jax
import jax.numpy as jnp
from jax import lax
from jax.experimental import pallas as pl
from jax.experimental.pallas import tpu as pltpu
from jax.experimental.pallas import tpu_sc as plsc

N = 10000
NP = 10240
E = 320000
D = 128
NC = 2
NS = 16
TILES = NC * NS
CH = 128
NCH = 80
GCH = 40
EP = TILES * NCH * CH
RPT = NP // NS
BR = 1000

_f32 = jnp.float32


def _mesh():
    return plsc.VectorSubcoreMesh(core_axis_name="c", subcore_axis_name="s",
                                  num_cores=NC, num_subcores=NS)


def _sc_deg_body(dst_hbm, out_hbm, hist, didx):
    c = lax.axis_index("c")
    s = lax.axis_index("s")
    t = c * NS + s
    ept = NCH * CH
    pltpu.sync_copy(dst_hbm.at[pl.ds(t * ept, ept)], didx)

    def zero(i, carry):
        hist[pl.ds(i * 16, 16)] = jnp.zeros((16,), _f32)
        return carry

    lax.fori_loop(0, NP // 16, zero, 0)

    def body(v, carry):
        idx = didx[pl.ds(v * 16, 16)]
        cnt, last = plsc.scan_count(idx)
        plsc.addupdate_scatter(hist, [idx], cnt.astype(_f32), mask=last)
        return carry

    lax.fori_loop(0, ept // 16, body, 0)
    pltpu.sync_copy(hist, out_hbm.at[pl.ds(t * NP, NP)])


def _sc_deg(dst_flat):
    return pl.kernel(
        _sc_deg_body,
        out_type=jax.ShapeDtypeStruct((TILES * NP,), _f32),
        mesh=_mesh(),
        compiler_params=pltpu.CompilerParams(needs_layout_passes=False),
        scratch_types=[
            pltpu.VMEM((NP,), _f32),
            pltpu.VMEM((NCH * CH,), jnp.int32),
        ],
    )(dst_flat)


def _sc_agg_body(hp_hbm, src_hbm, dst_hbm, zeros_hbm, out_hbm,
                 acc, sidx, didx, rows, gsem, ssem):
    c = lax.axis_index("c")
    s = lax.axis_index("s")
    t = c * NS + s
    r0 = s * RPT

    @pl.when(c == 0)
    def _():
        pltpu.sync_copy(hp_hbm.at[pl.ds(r0, RPT)], acc.at[pl.ds(r0, RPT)])

    @pl.when(c != 0)
    def _():
        pltpu.sync_copy(zeros_hbm.at[pl.ds(r0, RPT)], acc.at[pl.ds(r0, RPT)])

    plsc.subcore_barrier()

    for g in range(NCH // GCH):
        pltpu.sync_copy(src_hbm.at[pl.ds(t * NCH + g * GCH, GCH)], sidx)
        pltpu.sync_copy(dst_hbm.at[pl.ds(t * NCH + g * GCH, GCH)], didx)
        pltpu.async_copy(hp_hbm.at[sidx.at[0]], rows.at[0], gsem.at[0])

        def body(j, carry):
            b = lax.rem(j, 2)
            pltpu.make_async_copy(hp_hbm.at[sidx.at[j]], rows.at[b],
                                  gsem.at[b]).wait()

            @pl.when(j >= 1)
            def _():
                pltpu.make_async_copy(rows.at[1 - b],
                                      acc.at[didx.at[j - 1]],
                                      ssem.at[1 - b]).wait()

            @pl.when(j + 1 < GCH)
            def _():
                pltpu.async_copy(hp_hbm.at[sidx.at[j + 1]], rows.at[1 - b],
                                 gsem.at[1 - b])

            pltpu.async_copy(rows.at[b], acc.at[didx.at[j]], ssem.at[b],
                             add=True)
            return carry

        lax.fori_loop(0, GCH, body, 0)
        pltpu.make_async_copy(rows.at[1 - GCH % 2], acc.at[didx.at[GCH - 1]],
                              ssem.at[1 - GCH % 2]).wait()
    plsc.subcore_barrier()
    pltpu.sync_copy(acc.at[pl.ds(r0, RPT)], out_hbm.at[c, pl.ds(r0, RPT)])


def _sc_agg(hp, src2, dst2, zeros128):
    return pl.kernel(
        _sc_agg_body,
        out_type=jax.ShapeDtypeStruct((NC, NP, D), _f32),
        mesh=_mesh(),
        scratch_types=[
            pltpu.VMEM_SHARED((NP, D), _f32),
            pltpu.VMEM((GCH, CH), jnp.int32),
            pltpu.VMEM((GCH, CH), jnp.int32),
            pltpu.VMEM((2, CH, D), _f32),
            pltpu.SemaphoreType.DMA((2,)),
            pltpu.SemaphoreType.DMA((2,)),
        ],
    )(hp, src2, dst2, zeros128)


def _tc_degsum_body(h_ref, out_ref):
    out_ref[...] = jnp.sum(h_ref[...], axis=0, keepdims=True)


def _tc_degsum(hists):
    bc = 2048
    return pl.pallas_call(
        _tc_degsum_body,
        grid=(NP // bc,),
        in_specs=[pl.BlockSpec((TILES, bc), lambda i: (0, i))],
        out_specs=pl.BlockSpec((1, bc), lambda i: (0, i)),
        out_shape=jax.ShapeDtypeStruct((1, NP), _f32),
    )(hists)


def _tc_prep_body(x_ref, w_ref, deg_ref, hp_ref, dis_ref):
    dis = lax.rsqrt(deg_ref[...] + 1.0)
    h = jnp.dot(x_ref[...], w_ref[...], preferred_element_type=_f32)
    hp_ref[...] = h * dis
    dis_ref[...] = dis


def _tc_prep(x, w1, deg2):
    return pl.pallas_call(
        _tc_prep_body,
        grid=(N // BR,),
        in_specs=[
            pl.BlockSpec((BR, D), lambda i: (i, 0)),
            pl.BlockSpec((D, D), lambda i: (0, 0)),
            pl.BlockSpec((BR, 1), lambda i: (i, 0)),
        ],
        out_specs=[
            pl.BlockSpec((BR, D), lambda i: (i, 0)),
            pl.BlockSpec((BR, 1), lambda i: (i, 0)),
        ],
        out_shape=[
            jax.ShapeDtypeStruct((NP, D), _f32),
            jax.ShapeDtypeStruct((NP, 1), _f32),
        ],
    )(x, w1, deg2)


def _tc_mid_body(pa_ref, pb_ref, dis_ref, b_ref, w_ref, hp_ref):
    dis = dis_ref[...]
    xn = jnp.maximum((pa_ref[0] + pb_ref[0]) * dis + b_ref[...], 0.0)
    hp_ref[...] = jnp.dot(xn, w_ref[...], preferred_element_type=_f32) * dis


def _tc_mid(p, dis, b, w):
    return pl.pallas_call(
        _tc_mid_body,
        grid=(N // BR,),
        in_specs=[
            pl.BlockSpec((1, BR, D), lambda i: (0, i, 0)),
            pl.BlockSpec((1, BR, D), lambda i: (1, i, 0)),
            pl.BlockSpec((BR, 1), lambda i: (i, 0)),
            pl.BlockSpec((1, D), lambda i: (0, 0)),
            pl.BlockSpec((D, D), lambda i: (0, 0)),
        ],
        out_specs=pl.BlockSpec((BR, D), lambda i: (i, 0)),
        out_shape=jax.ShapeDtypeStruct((NP, D), _f32),
    )(p, p, dis, b, w)


def _tc_fin_body(pa_ref, pb_ref, dis_ref, b_ref, out_ref):
    out_ref[...] = (pa_ref[0] + pb_ref[0]) * dis_ref[...] + b_ref[...]


def _tc_fin(p, dis, b):
    return pl.pallas_call(
        _tc_fin_body,
        grid=(N // BR,),
        in_specs=[
            pl.BlockSpec((1, BR, D), lambda i: (0, i, 0)),
            pl.BlockSpec((1, BR, D), lambda i: (1, i, 0)),
            pl.BlockSpec((BR, 1), lambda i: (i, 0)),
            pl.BlockSpec((1, D), lambda i: (0, 0)),
        ],
        out_specs=pl.BlockSpec((BR, D), lambda i: (i, 0)),
        out_shape=jax.ShapeDtypeStruct((N, D), _f32),
    )(p, p, dis, b)


def kernel(x, edge_index, W1, b1, W2, b2, W3, b3):
    src = edge_index[0].astype(jnp.int32)
    dst = edge_index[1].astype(jnp.int32)
    pad = N + (jnp.arange(EP - E, dtype=jnp.int32) % (NP - N))
    src_flat = jnp.concatenate([src, pad])
    dst_flat = jnp.concatenate([dst, pad])
    src2 = src_flat.reshape(TILES * NCH, CH)
    dst2 = dst_flat.reshape(TILES * NCH, CH)

    zeros128 = jnp.zeros((NP, D), _f32)
    b1r = b1.reshape(1, D)
    b2r = b2.reshape(1, D)
    b3r = b3.reshape(1, D)

    hists = _sc_deg(dst_flat)
    deg2 = _tc_degsum(hists.reshape(TILES, NP)).reshape(NP, 1)
    hp1, dis = _tc_prep(x, W1, deg2)
    p1 = _sc_agg(hp1, src2, dst2, zeros128)
    hp2 = _tc_mid(p1, dis, b1r, W2)
    p2 = _sc_agg(hp2, src2, dst2, zeros128)
    hp3 = _tc_mid(p2, dis, b2r, W3)
    p3 = _sc_agg(hp3, src2, dst2, zeros128)
    return _tc_fin(p3, dis, b3r)


# speedup vs baseline: 1.2960x; 1.0010x over previous
import jax
import jax.numpy as jnp
from jax import lax
from jax.experimental import pallas as pl
from jax.experimental.pallas import tpu as pltpu
from jax.experimental.pallas import tpu_sc as plsc

N = 10000
NP = 10240
E = 320000
D = 128
NC = 2
NS = 16
TILES = NC * NS
CH = 128
NCH = 80
GCH = 40
EP = TILES * NCH * CH
RPT = NP // NS
BR = 1000

_f32 = jnp.float32


def _mesh():
    return plsc.VectorSubcoreMesh(core_axis_name="c", subcore_axis_name="s",
                                  num_cores=NC, num_subcores=NS)


def _sc_deg_body(dst_hbm, out_hbm, hist, didx):
    c = lax.axis_index("c")
    s = lax.axis_index("s")
    t = c * NS + s
    ept = NCH * CH
    pltpu.sync_copy(dst_hbm.at[pl.ds(t * ept, ept)], didx)

    def zero(i, carry):
        hist[pl.ds(i * 16, 16)] = jnp.zeros((16,), _f32)
        return carry

    lax.fori_loop(0, NP // 16, zero, 0)

    def body(v, carry):
        idx = didx[pl.ds(v * 16, 16)]
        cnt, last = plsc.scan_count(idx)
        plsc.addupdate_scatter(hist, [idx], cnt.astype(_f32), mask=last)
        return carry

    lax.fori_loop(0, ept // 16, body, 0)
    pltpu.sync_copy(hist, out_hbm.at[pl.ds(t * NP, NP)])


def _sc_deg(dst_flat):
    return pl.kernel(
        _sc_deg_body,
        out_type=jax.ShapeDtypeStruct((TILES * NP,), _f32),
        mesh=_mesh(),
        compiler_params=pltpu.CompilerParams(needs_layout_passes=False),
        scratch_types=[
            pltpu.VMEM((NP,), _f32),
            pltpu.VMEM((NCH * CH,), jnp.int32),
        ],
    )(dst_flat)


def _sc_agg_body(hp_hbm, src_hbm, dst_hbm, zeros_hbm, out_hbm,
                 acc, sidx, didx, rows, gsem, ssem):
    c = lax.axis_index("c")
    s = lax.axis_index("s")
    t = c * NS + s
    r0 = s * RPT

    @pl.when(c == 0)
    def _():
        pltpu.sync_copy(hp_hbm.at[pl.ds(r0, RPT)], acc.at[pl.ds(r0, RPT)])

    @pl.when(c != 0)
    def _():
        pltpu.sync_copy(zeros_hbm.at[pl.ds(r0, RPT)], acc.at[pl.ds(r0, RPT)])

    plsc.subcore_barrier()

    for g in range(NCH // GCH):
        pltpu.sync_copy(src_hbm.at[pl.ds(t * NCH + g * GCH, GCH)], sidx)
        pltpu.sync_copy(dst_hbm.at[pl.ds(t * NCH + g * GCH, GCH)], didx)
        pltpu.async_copy(hp_hbm.at[sidx.at[0]], rows.at[0], gsem.at[0])

        def body(j, carry):
            b = lax.rem(j, 2)
            pltpu.make_async_copy(hp_hbm.at[sidx.at[j]], rows.at[b],
                                  gsem.at[b]).wait()

            @pl.when(j >= 1)
            def _():
                pltpu.make_async_copy(rows.at[1 - b],
                                      acc.at[didx.at[j - 1]],
                                      ssem.at[1 - b]).wait()

            @pl.when(j + 1 < GCH)
            def _():
                pltpu.async_copy(hp_hbm.at[sidx.at[j + 1]], rows.at[1 - b],
                                 gsem.at[1 - b])

            pltpu.async_copy(rows.at[b], acc.at[didx.at[j]], ssem.at[b],
                             add=True)
            return carry

        lax.fori_loop(0, GCH, body, 0)
        pltpu.make_async_copy(rows.at[1 - GCH % 2], acc.at[didx.at[GCH - 1]],
                              ssem.at[1 - GCH % 2]).wait()
    plsc.subcore_barrier()
    pltpu.sync_copy(acc.at[pl.ds(r0, RPT)], out_hbm.at[c, pl.ds(r0, RPT)])


def _sc_agg(hp, src2, dst2, zeros128):
    return pl.kernel(
        _sc_agg_body,
        out_type=jax.ShapeDtypeStruct((NC, NP, D), _f32),
        mesh=_mesh(),
        scratch_types=[
            pltpu.VMEM_SHARED((NP, D), _f32),
            pltpu.VMEM((GCH, CH), jnp.int32),
            pltpu.VMEM((GCH, CH), jnp.int32),
            pltpu.VMEM((2, CH, D), _f32),
            pltpu.SemaphoreType.DMA((2,)),
            pltpu.SemaphoreType.DMA((2,)),
        ],
    )(hp, src2, dst2, zeros128)


def _tc_degsum_body(h_ref, out_ref):
    out_ref[...] = jnp.sum(h_ref[...], axis=0, keepdims=True)


def _tc_degsum(hists):
    bc = 2048
    return pl.pallas_call(
        _tc_degsum_body,
        grid=(NP // bc,),
        in_specs=[pl.BlockSpec((TILES, bc), lambda i: (0, i))],
        out_specs=pl.BlockSpec((1, bc), lambda i: (0, i)),
        out_shape=jax.ShapeDtypeStruct((1, NP), _f32),
    )(hists)


def _tc_mm_body(x_ref, w_ref, h_ref):
    h_ref[...] = jnp.dot(x_ref[...], w_ref[...], preferred_element_type=_f32)


def _tc_mm(x, w1):
    return pl.pallas_call(
        _tc_mm_body,
        grid=(N // BR,),
        in_specs=[
            pl.BlockSpec((BR, D), lambda i: (i, 0)),
            pl.BlockSpec((D, D), lambda i: (0, 0)),
        ],
        out_specs=pl.BlockSpec((BR, D), lambda i: (i, 0)),
        out_shape=jax.ShapeDtypeStruct((N, D), _f32),
    )(x, w1)


def _tc_prep_body(h_ref, deg_ref, hp_ref, dis_ref):
    dis = lax.rsqrt(deg_ref[...] + 1.0)
    hp_ref[...] = h_ref[...] * dis
    dis_ref[...] = dis


def _tc_prep(h1, deg2):
    return pl.pallas_call(
        _tc_prep_body,
        grid=(N // BR,),
        in_specs=[
            pl.BlockSpec((BR, D), lambda i: (i, 0)),
            pl.BlockSpec((BR, 1), lambda i: (i, 0)),
        ],
        out_specs=[
            pl.BlockSpec((BR, D), lambda i: (i, 0)),
            pl.BlockSpec((BR, 1), lambda i: (i, 0)),
        ],
        out_shape=[
            jax.ShapeDtypeStruct((NP, D), _f32),
            jax.ShapeDtypeStruct((NP, 1), _f32),
        ],
    )(h1, deg2)


def _tc_mid_body(pa_ref, pb_ref, dis_ref, b_ref, w_ref, hp_ref):
    dis = dis_ref[...]
    xn = jnp.maximum((pa_ref[0] + pb_ref[0]) * dis + b_ref[...], 0.0)
    hp_ref[...] = jnp.dot(xn, w_ref[...], preferred_element_type=_f32) * dis


def _tc_mid(p, dis, b, w):
    return pl.pallas_call(
        _tc_mid_body,
        grid=(N // BR,),
        in_specs=[
            pl.BlockSpec((1, BR, D), lambda i: (0, i, 0)),
            pl.BlockSpec((1, BR, D), lambda i: (1, i, 0)),
            pl.BlockSpec((BR, 1), lambda i: (i, 0)),
            pl.BlockSpec((1, D), lambda i: (0, 0)),
            pl.BlockSpec((D, D), lambda i: (0, 0)),
        ],
        out_specs=pl.BlockSpec((BR, D), lambda i: (i, 0)),
        out_shape=jax.ShapeDtypeStruct((NP, D), _f32),
    )(p, p, dis, b, w)


def _tc_fin_body(pa_ref, pb_ref, dis_ref, b_ref, out_ref):
    out_ref[...] = (pa_ref[0] + pb_ref[0]) * dis_ref[...] + b_ref[...]


def _tc_fin(p, dis, b):
    return pl.pallas_call(
        _tc_fin_body,
        grid=(N // BR,),
        in_specs=[
            pl.BlockSpec((1, BR, D), lambda i: (0, i, 0)),
            pl.BlockSpec((1, BR, D), lambda i: (1, i, 0)),
            pl.BlockSpec((BR, 1), lambda i: (i, 0)),
            pl.BlockSpec((1, D), lambda i: (0, 0)),
        ],
        out_specs=pl.BlockSpec((BR, D), lambda i: (i, 0)),
        out_shape=jax.ShapeDtypeStruct((N, D), _f32),
    )(p, p, dis, b)


def kernel(x, edge_index, W1, b1, W2, b2, W3, b3):
    src = edge_index[0].astype(jnp.int32)
    dst = edge_index[1].astype(jnp.int32)
    pad = N + (jnp.arange(EP - E, dtype=jnp.int32) % (NP - N))
    src_flat = jnp.concatenate([src, pad])
    dst_flat = jnp.concatenate([dst, pad])
    src2 = src_flat.reshape(TILES * NCH, CH)
    dst2 = dst_flat.reshape(TILES * NCH, CH)

    zeros128 = jnp.zeros((NP, D), _f32)
    b1r = b1.reshape(1, D)
    b2r = b2.reshape(1, D)
    b3r = b3.reshape(1, D)

    h1 = _tc_mm(x, W1)
    hists = _sc_deg(dst_flat)
    deg2 = _tc_degsum(hists.reshape(TILES, NP)).reshape(NP, 1)
    hp1, dis = _tc_prep(h1, deg2)
    p1 = _sc_agg(hp1, src2, dst2, zeros128)
    hp2 = _tc_mid(p1, dis, b1r, W2)
    p2 = _sc_agg(hp2, src2, dst2, zeros128)
    hp3 = _tc_mid(p2, dis, b2r, W3)
    p3 = _sc_agg(hp3, src2, dst2, zeros128)
    return _tc_fin(p3, dis, b3r)
